# baseline (speedup 1.0000x reference)
import functools

import jax
import jax.numpy as jnp
from jax import lax
from jax.experimental import pallas as pl
from jax.experimental.pallas import tpu as pltpu
from jax.experimental.pallas import tpu_sc as plsc

BATCH = 4096
MAX_LEN = 200
EMB = 128
EPS = 1e-8
TABLE_ROWS = 128
PAD_ID = TABLE_ROWS - 1

_R = 32


def _tc_body(tokens_ref, lens_ref, dec_ref, mtok_ref, ts_ref):
    i = pl.program_id(0)
    lens = lens_ref[...]
    toks = tokens_ref[...]
    pos = lax.broadcasted_iota(jnp.int32, (_R, MAX_LEN), 1)
    mask = pos < lens
    mtok_ref[...] = jnp.where(mask, toks, PAD_ID)
    maskf = mask.astype(jnp.float32) + EPS
    dec_ref[...] = jnp.broadcast_to(maskf[:, :, None], (_R, MAX_LEN, EMB))
    local_max = jnp.max(lens)

    @pl.when(i == 0)
    def _init():
        ts_ref[0] = local_max

    @pl.when(i > 0)
    def _acc():
        ts_ref[0] = jnp.maximum(ts_ref[0], local_max)


_tc_call = pl.pallas_call(
    _tc_body,
    grid=(BATCH // _R,),
    in_specs=[
        pl.BlockSpec((_R, MAX_LEN), lambda i: (i, 0)),
        pl.BlockSpec((_R, 1), lambda i: (i, 0)),
    ],
    out_specs=[
        pl.BlockSpec((_R, MAX_LEN, EMB), lambda i: (i, 0, 0)),
        pl.BlockSpec((_R, MAX_LEN), lambda i: (i, 0)),
        pl.BlockSpec(memory_space=pltpu.SMEM),
    ],
    out_shape=[
        jax.ShapeDtypeStruct((BATCH, MAX_LEN, EMB), jnp.float32),
        jax.ShapeDtypeStruct((BATCH, MAX_LEN), jnp.int32),
        jax.ShapeDtypeStruct((1,), jnp.int32),
    ],
)

_NC, _NS = 2, 16
_NW = _NC * _NS
_B = BATCH * MAX_LEN
_BPW = _B // _NW
_CH = 128
_NCHUNK = _BPW // _CH

@functools.cache
def _make_sc_gather():
    mesh = plsc.VectorSubcoreMesh(core_axis_name="c", subcore_axis_name="s")

    @functools.partial(
        pl.kernel,
        mesh=mesh,
        out_type=jax.ShapeDtypeStruct((_B, EMB), jnp.float32),
        scratch_types=[
            pltpu.VMEM((_NCHUNK, _CH), jnp.int32),
            pltpu.VMEM((_CH, EMB), jnp.float32),
            pltpu.SemaphoreType.DMA,
        ],
    )
    def _sc_gather(table_hbm, idx_hbm, out_hbm, idx_v, rows_v, sem):
        wid = lax.axis_index("s") * _NC + lax.axis_index("c")
        base = wid * _BPW
        pltpu.sync_copy(idx_hbm.at[pl.ds(wid * _NCHUNK, _NCHUNK)], idx_v)

        def body(j, carry):
            pltpu.async_copy(table_hbm.at[idx_v.at[j]], rows_v, sem).wait()
            pltpu.sync_copy(rows_v, out_hbm.at[pl.ds(base + j * _CH, _CH)])
            return carry

        lax.fori_loop(0, _NCHUNK, body, 0)

    return _sc_gather


def kernel(tokens, seq_lengths, embeddings):
    pad = TABLE_ROWS - embeddings.shape[0]
    table = jnp.concatenate(
        [embeddings, jnp.zeros((pad, EMB), jnp.float32)], axis=0
    )
    lens2d = seq_lengths.reshape(BATCH, 1)
    dec_mask, mtok, ts = _tc_call(tokens, lens2d)
    idx2d = mtok.reshape(_NW * _NCHUNK, _CH)
    x = _make_sc_gather()(table, idx2d).reshape(BATCH, MAX_LEN, EMB)
    return x, dec_mask, ts[0]


# speedup vs baseline: 34.5117x; 34.5117x over previous
import functools

import jax
import jax.numpy as jnp
from jax import lax
from jax.experimental import pallas as pl
from jax.experimental.pallas import tpu as pltpu
from jax.experimental.pallas import tpu_sc as plsc

BATCH = 4096
MAX_LEN = 200
EMB = 128
EPS = 1e-8
TABLE_ROWS = 128
PAD_ID = TABLE_ROWS - 1

_R = 32


def _tc_body(tokens_ref, lens_ref, dec_ref, mtok_ref, ts_ref):
    i = pl.program_id(0)
    lens = lens_ref[...]
    toks = tokens_ref[...]
    pos = lax.broadcasted_iota(jnp.int32, (_R, MAX_LEN), 1)
    mask = pos < lens
    mtok_ref[...] = jnp.where(mask, toks, PAD_ID)
    maskf = mask.astype(jnp.float32) + EPS
    dec_ref[...] = jnp.broadcast_to(maskf[:, :, None], (_R, MAX_LEN, EMB))
    local_max = jnp.max(lens)

    @pl.when(i == 0)
    def _init():
        ts_ref[0] = local_max

    @pl.when(i > 0)
    def _acc():
        ts_ref[0] = jnp.maximum(ts_ref[0], local_max)


_tc_call = pl.pallas_call(
    _tc_body,
    grid=(BATCH // _R,),
    in_specs=[
        pl.BlockSpec((_R, MAX_LEN), lambda i: (i, 0)),
        pl.BlockSpec((_R, 1), lambda i: (i, 0)),
    ],
    out_specs=[
        pl.BlockSpec((_R, MAX_LEN, EMB), lambda i: (i, 0, 0)),
        pl.BlockSpec((_R, MAX_LEN), lambda i: (i, 0)),
        pl.BlockSpec(memory_space=pltpu.SMEM),
    ],
    out_shape=[
        jax.ShapeDtypeStruct((BATCH, MAX_LEN, EMB), jnp.float32),
        jax.ShapeDtypeStruct((BATCH, MAX_LEN), jnp.int32),
        jax.ShapeDtypeStruct((1,), jnp.int32),
    ],
)

_NC, _NS = 2, 16
_NW = _NC * _NS
_B = BATCH * MAX_LEN
_BPW = _B // _NW
_CH = 128
_NCHUNK = _BPW // _CH

@functools.cache
def _make_sc_gather():
    mesh = plsc.VectorSubcoreMesh(core_axis_name="c", subcore_axis_name="s")

    @functools.partial(
        pl.kernel,
        mesh=mesh,
        out_type=jax.ShapeDtypeStruct((_B, EMB), jnp.float32),
        scratch_types=[
            pltpu.VMEM((_NCHUNK, _CH), jnp.int32),
            pltpu.VMEM((_CH, EMB), jnp.float32),
            pltpu.VMEM((_CH, EMB), jnp.float32),
            pltpu.VMEM_SHARED((TABLE_ROWS, EMB), jnp.float32),
            pltpu.SemaphoreType.DMA,
            pltpu.SemaphoreType.DMA,
            pltpu.SemaphoreType.DMA,
            pltpu.SemaphoreType.DMA,
        ],
    )
    def _sc_gather(
        table_hbm, idx_hbm, out_hbm,
        idx_v, buf0, buf1, table_sh, gsem0, gsem1, ssem0, ssem1,
    ):
        cid = lax.axis_index("c")
        sid = lax.axis_index("s")
        wid = sid * _NC + cid
        base = wid * _BPW

        @pl.when(sid == 0)
        def _stage_table():
            pltpu.sync_copy(table_hbm, table_sh)

        pltpu.sync_copy(idx_hbm.at[pl.ds(wid * _NCHUNK, _NCHUNK)], idx_v)
        plsc.subcore_barrier()

        def gather(j, buf, sem):
            return pltpu.make_async_copy(table_sh.at[idx_v.at[j]], buf, sem)

        def scatter(j, buf, sem):
            return pltpu.make_async_copy(
                buf, out_hbm.at[pl.ds(base + j * _CH, _CH)], sem
            )

        g0 = gather(0, buf0, gsem0)
        g0.start()
        g1 = gather(1, buf1, gsem1)
        g1.start()

        def body(p, carry):
            j0 = 2 * p
            j1 = j0 + 1
            j2 = lax.rem(j0 + 2, _NCHUNK)
            j3 = lax.rem(j0 + 3, _NCHUNK)
            gather(j0, buf0, gsem0).wait()
            s0 = scatter(j0, buf0, ssem0)
            s0.start()
            gather(j1, buf1, gsem1).wait()
            s1 = scatter(j1, buf1, ssem1)
            s1.start()
            s0.wait()
            gather(j2, buf0, gsem0).start()
            s1.wait()
            gather(j3, buf1, gsem1).start()
            return carry

        lax.fori_loop(0, _NCHUNK // 2, body, 0)
        gather(0, buf0, gsem0).wait()
        gather(1, buf1, gsem1).wait()

    return _sc_gather


def kernel(tokens, seq_lengths, embeddings):
    pad = TABLE_ROWS - embeddings.shape[0]
    table = jnp.concatenate(
        [embeddings, jnp.zeros((pad, EMB), jnp.float32)], axis=0
    )
    lens2d = seq_lengths.reshape(BATCH, 1)
    dec_mask, mtok, ts = _tc_call(tokens, lens2d)
    idx2d = mtok.reshape(_NW * _NCHUNK, _CH)
    x = _make_sc_gather()(table, idx2d).reshape(BATCH, MAX_LEN, EMB)
    return x, dec_mask, ts[0]


# speedup vs baseline: 54.7586x; 1.5867x over previous
import functools

import jax
import jax.numpy as jnp
from jax import lax
from jax.experimental import pallas as pl
from jax.experimental.pallas import tpu as pltpu
from jax.experimental.pallas import tpu_sc as plsc

BATCH = 4096
MAX_LEN = 200
EMB = 128
EPS = 1e-8
TABLE_ROWS = 128
PAD_ID = TABLE_ROWS - 1

_R = 32


def _tc_dec_body(lens_ref, dec_ref):
    lens = lens_ref[...]
    pos = lax.broadcasted_iota(jnp.int32, (_R, MAX_LEN, EMB), 1)
    mask = pos < lens[:, :, None]
    dec_ref[...] = mask.astype(jnp.float32) + EPS


_tc_dec_call = pl.pallas_call(
    _tc_dec_body,
    grid=(BATCH // _R,),
    in_specs=[
        pl.BlockSpec((_R, 1), lambda i: (i, 0)),
    ],
    out_specs=[
        pl.BlockSpec((_R, MAX_LEN, EMB), lambda i: (i, 0, 0)),
    ],
    out_shape=[
        jax.ShapeDtypeStruct((BATCH, MAX_LEN, EMB), jnp.float32),
    ],
)

_RM = 512


def _tc_mask_body(tokens_ref, lens_ref, mtok_ref, ts_ref):
    i = pl.program_id(0)
    lens = lens_ref[...]
    toks = tokens_ref[...]
    pos = lax.broadcasted_iota(jnp.int32, (_RM, MAX_LEN), 1)
    mtok_ref[...] = jnp.where(pos < lens, toks, PAD_ID)
    local_max = jnp.max(lens)

    @pl.when(i == 0)
    def _init():
        ts_ref[0] = local_max

    @pl.when(i > 0)
    def _acc():
        ts_ref[0] = jnp.maximum(ts_ref[0], local_max)


_tc_mask_call = pl.pallas_call(
    _tc_mask_body,
    grid=(BATCH // _RM,),
    in_specs=[
        pl.BlockSpec((_RM, MAX_LEN), lambda i: (i, 0)),
        pl.BlockSpec((_RM, 1), lambda i: (i, 0)),
    ],
    out_specs=[
        pl.BlockSpec((_RM, MAX_LEN), lambda i: (i, 0)),
        pl.BlockSpec(memory_space=pltpu.SMEM),
    ],
    out_shape=[
        jax.ShapeDtypeStruct((BATCH, MAX_LEN), jnp.int32),
        jax.ShapeDtypeStruct((1,), jnp.int32),
    ],
)

_NC, _NS = 2, 16
_NW = _NC * _NS
_B = BATCH * MAX_LEN
_BPW = _B // _NW
_CH = 128
_NCHUNK = _BPW // _CH
_RPW = BATCH // _NW

@functools.cache
def _make_sc_gather():
    mesh = plsc.VectorSubcoreMesh(core_axis_name="c", subcore_axis_name="s")

    @functools.partial(
        pl.kernel,
        mesh=mesh,
        out_type=jax.ShapeDtypeStruct((_B, EMB), jnp.float32),
        scratch_types=[
            pltpu.VMEM((_NCHUNK, _CH), jnp.int32),
            pltpu.VMEM((_CH, EMB), jnp.float32),
            pltpu.VMEM((_CH, EMB), jnp.float32),
            pltpu.VMEM_SHARED((TABLE_ROWS, EMB), jnp.float32),
            pltpu.SemaphoreType.DMA,
            pltpu.SemaphoreType.DMA,
            pltpu.SemaphoreType.DMA,
            pltpu.SemaphoreType.DMA,
        ],
    )
    def _sc_gather(
        table_hbm, idx_hbm, out_hbm,
        idx_v, buf0, buf1, table_sh, gsem0, gsem1, ssem0, ssem1,
    ):
        cid = lax.axis_index("c")
        sid = lax.axis_index("s")
        wid = sid * _NC + cid
        base = wid * _BPW

        @pl.when(sid == 0)
        def _stage_table():
            pltpu.sync_copy(table_hbm, table_sh)

        pltpu.sync_copy(idx_hbm.at[pl.ds(wid * _NCHUNK, _NCHUNK)], idx_v)
        plsc.subcore_barrier()

        def gather(j, buf, sem):
            return pltpu.make_async_copy(table_sh.at[idx_v.at[j]], buf, sem)

        def scatter(j, buf, sem):
            return pltpu.make_async_copy(
                buf, out_hbm.at[pl.ds(base + j * _CH, _CH)], sem
            )

        g0 = gather(0, buf0, gsem0)
        g0.start()
        g1 = gather(1, buf1, gsem1)
        g1.start()

        def body(p, carry):
            j0 = 2 * p
            j1 = j0 + 1
            j2 = lax.rem(j0 + 2, _NCHUNK)
            j3 = lax.rem(j0 + 3, _NCHUNK)
            gather(j0, buf0, gsem0).wait()
            s0 = scatter(j0, buf0, ssem0)
            s0.start()
            gather(j1, buf1, gsem1).wait()
            s1 = scatter(j1, buf1, ssem1)
            s1.start()
            s0.wait()
            gather(j2, buf0, gsem0).start()
            s1.wait()
            gather(j3, buf1, gsem1).start()
            return carry

        lax.fori_loop(0, _NCHUNK // 2, body, 0)
        gather(0, buf0, gsem0).wait()
        gather(1, buf1, gsem1).wait()

    return _sc_gather


def kernel(tokens, seq_lengths, embeddings):
    pad = TABLE_ROWS - embeddings.shape[0]
    table = jnp.concatenate(
        [embeddings, jnp.zeros((pad, EMB), jnp.float32)], axis=0
    )
    lens2d = seq_lengths.reshape(BATCH, 1)
    mtok, ts = _tc_mask_call(tokens, lens2d)
    idx2d = mtok.reshape(_NW * _NCHUNK, _CH)
    x = _make_sc_gather()(table, idx2d)
    x = x.reshape(BATCH, MAX_LEN, EMB)
    dec_mask, = _tc_dec_call(lens2d)
    return x, dec_mask, ts[0]
